# initial kernel scaffold (unmeasured)
import jax
import jax.numpy as jnp
from jax import lax
from jax.experimental import pallas as pl
from jax.experimental.pallas import tpu as pltpu

T = 4096
D = 2048
F = 4096
E_LOCAL = 4
N_TOK = 2 * T
C = 1152
F_TILE = 512
A_ROWS = 32


def _neighbor():
    my_x = lax.axis_index("x")
    my_y = lax.axis_index("y")
    my_z = lax.axis_index("z")
    return my_y, (my_x, 1 - my_y, my_z)


def _exchange_body(x_ref, a_ref, xall_ref, aall_ref,
                   sx_send, sx_recv, sa_send, sa_recv):
    my_y, nbr = _neighbor()

    barrier = pltpu.get_barrier_semaphore()
    pl.semaphore_signal(barrier, inc=1, device_id=nbr,
                        device_id_type=pl.DeviceIdType.MESH)
    pl.semaphore_wait(barrier, 1)

    row0 = my_y * T
    arow0 = my_y * A_ROWS
    xall_ref[pl.ds(row0, T), :] = x_ref[...]
    aall_ref[pl.ds(arow0, A_ROWS), :] = a_ref[...]

    rx = pltpu.make_async_remote_copy(
        src_ref=xall_ref.at[pl.ds(row0, T)],
        dst_ref=xall_ref.at[pl.ds(row0, T)],
        send_sem=sx_send, recv_sem=sx_recv,
        device_id=nbr, device_id_type=pl.DeviceIdType.MESH,
    )
    ra = pltpu.make_async_remote_copy(
        src_ref=aall_ref.at[pl.ds(arow0, A_ROWS)],
        dst_ref=aall_ref.at[pl.ds(arow0, A_ROWS)],
        send_sem=sa_send, recv_sem=sa_recv,
        device_id=nbr, device_id_type=pl.DeviceIdType.MESH,
    )
    rx.start()
    ra.start()
    rx.wait()
    ra.wait()


def _exchange(x_bf16, assign2d):
    return pl.pallas_call(
        _exchange_body,
        out_shape=(
            jax.ShapeDtypeStruct((N_TOK, D), jnp.bfloat16),
            jax.ShapeDtypeStruct((2 * A_ROWS, 128), jnp.int32),
        ),
        in_specs=[
            pl.BlockSpec(memory_space=pltpu.VMEM),
            pl.BlockSpec(memory_space=pltpu.VMEM),
        ],
        out_specs=(
            pl.BlockSpec(memory_space=pltpu.VMEM),
            pl.BlockSpec(memory_space=pltpu.VMEM),
        ),
        scratch_shapes=[
            pltpu.SemaphoreType.DMA,
            pltpu.SemaphoreType.DMA,
            pltpu.SemaphoreType.DMA,
            pltpu.SemaphoreType.DMA,
        ],
        compiler_params=pltpu.CompilerParams(collective_id=0),
    )(x_bf16, assign2d)


def _moe_body(xg_ref, w1_ref, w2_ref, og_ref, acc_ref):
    fb = pl.program_id(1)
    n_fb = pl.num_programs(1)
    xg = xg_ref[0]
    w1 = w1_ref[0].astype(jnp.bfloat16)
    h = jnp.maximum(
        jnp.dot(xg, w1, preferred_element_type=jnp.float32), 0.0
    ).astype(jnp.bfloat16)
    w2 = w2_ref[0].astype(jnp.bfloat16)
    p = jnp.dot(h, w2, preferred_element_type=jnp.float32)

    @pl.when(fb == 0)
    def _():
        acc_ref[...] = p

    @pl.when(fb > 0)
    def _():
        acc_ref[...] += p

    @pl.when(fb == n_fb - 1)
    def _():
        og_ref[0] = acc_ref[...].astype(jnp.bfloat16)


def _moe(xg, w1, w2):
    n_fb = F // F_TILE
    return pl.pallas_call(
        _moe_body,
        grid=(E_LOCAL, n_fb),
        out_shape=jax.ShapeDtypeStruct((E_LOCAL, C, D), jnp.bfloat16),
        in_specs=[
            pl.BlockSpec((1, C, D), lambda e, fb: (e, 0, 0)),
            pl.BlockSpec((1, D, F_TILE), lambda e, fb: (e, 0, fb)),
            pl.BlockSpec((1, F_TILE, D), lambda e, fb: (e, fb, 0)),
        ],
        out_specs=pl.BlockSpec((1, C, D), lambda e, fb: (e, 0, 0)),
        scratch_shapes=[pltpu.VMEM((C, D), jnp.float32)],
    )(xg, w1, w2)


def _combine_body(in_ref, out_ref, comm_ref, s_send, s_recv):
    my_y, nbr = _neighbor()

    barrier = pltpu.get_barrier_semaphore()
    pl.semaphore_signal(barrier, inc=1, device_id=nbr,
                        device_id_type=pl.DeviceIdType.MESH)
    pl.semaphore_wait(barrier, 1)

    other0 = (1 - my_y) * T
    r = pltpu.make_async_remote_copy(
        src_ref=in_ref.at[pl.ds(other0, T)],
        dst_ref=comm_ref,
        send_sem=s_send, recv_sem=s_recv,
        device_id=nbr, device_id_type=pl.DeviceIdType.MESH,
    )
    r.start()
    r.wait()

    mine0 = my_y * T
    out_ref[...] = in_ref[pl.ds(mine0, T), :] + comm_ref[...]


def _combine(out_all):
    return pl.pallas_call(
        _combine_body,
        out_shape=jax.ShapeDtypeStruct((T, D), jnp.bfloat16),
        in_specs=[pl.BlockSpec(memory_space=pltpu.VMEM)],
        out_specs=pl.BlockSpec(memory_space=pltpu.VMEM),
        scratch_shapes=[
            pltpu.VMEM((T, D), jnp.bfloat16),
            pltpu.SemaphoreType.DMA,
            pltpu.SemaphoreType.DMA,
        ],
        compiler_params=pltpu.CompilerParams(collective_id=1),
    )(out_all)


def kernel(x, assign, W1, W2):
    x_all, a2d = _exchange(x.astype(jnp.bfloat16), assign.reshape(A_ROWS, 128))
    assign_all = a2d.reshape(N_TOK)

    my_y = lax.axis_index("y")
    sort_idx = jnp.argsort(assign_all)
    sorted_a = assign_all[sort_idx]
    starts = jnp.searchsorted(sorted_a, jnp.arange(9, dtype=assign_all.dtype))
    e_ids = my_y * E_LOCAL + jnp.arange(E_LOCAL)
    start_e = starts[e_ids]
    end_e = starts[e_ids + 1]
    pos = start_e[:, None] + jnp.arange(C)[None, :]
    mask = pos < end_e[:, None]
    idx = sort_idx[jnp.minimum(pos, N_TOK - 1)]
    xg = x_all[idx]

    og = _moe(xg, W1, W2)

    flat_idx = jnp.where(mask, idx, N_TOK).reshape(-1)
    out_all = (
        jnp.zeros((N_TOK, D), jnp.bfloat16)
        .at[flat_idx]
        .set(og.reshape(-1, D), mode="drop")
    )

    out = _combine(out_all)
    return out.astype(jnp.float32)


# baseline (device time: 916545 ns/iter reference)
import jax
import jax.numpy as jnp
from jax import lax
from jax.experimental import pallas as pl
from jax.experimental.pallas import tpu as pltpu

T = 4096
D = 2048
F = 4096
E_LOCAL = 4
N_TOK = 2 * T
C = 1152
F_TILE = 512
A_ROWS = 32
VMEM_LIMIT = 60 * 1024 * 1024


def _neighbor():
    my_x = lax.axis_index("x")
    my_y = lax.axis_index("y")
    my_z = lax.axis_index("z")
    return my_y, (my_x, 1 - my_y, my_z)


def _exchange_body(x_ref, a_ref, xall_ref, aall_ref,
                   sx_send, sx_recv, sa_send, sa_recv):
    my_y, nbr = _neighbor()

    barrier = pltpu.get_barrier_semaphore()
    pl.semaphore_signal(barrier, inc=1, device_id=nbr,
                        device_id_type=pl.DeviceIdType.MESH)
    pl.semaphore_wait(barrier, 1)

    row0 = my_y * T
    arow0 = my_y * A_ROWS
    xall_ref[pl.ds(row0, T), :] = x_ref[...]
    aall_ref[pl.ds(arow0, A_ROWS), :] = a_ref[...]

    rx = pltpu.make_async_remote_copy(
        src_ref=xall_ref.at[pl.ds(row0, T)],
        dst_ref=xall_ref.at[pl.ds(row0, T)],
        send_sem=sx_send, recv_sem=sx_recv,
        device_id=nbr, device_id_type=pl.DeviceIdType.MESH,
    )
    ra = pltpu.make_async_remote_copy(
        src_ref=aall_ref.at[pl.ds(arow0, A_ROWS)],
        dst_ref=aall_ref.at[pl.ds(arow0, A_ROWS)],
        send_sem=sa_send, recv_sem=sa_recv,
        device_id=nbr, device_id_type=pl.DeviceIdType.MESH,
    )
    rx.start()
    ra.start()
    rx.wait()
    ra.wait()


def _exchange(x_bf16, assign2d):
    return pl.pallas_call(
        _exchange_body,
        out_shape=(
            jax.ShapeDtypeStruct((N_TOK, D), jnp.bfloat16),
            jax.ShapeDtypeStruct((2 * A_ROWS, 128), jnp.int32),
        ),
        in_specs=[
            pl.BlockSpec(memory_space=pltpu.VMEM),
            pl.BlockSpec(memory_space=pltpu.VMEM),
        ],
        out_specs=(
            pl.BlockSpec(memory_space=pltpu.VMEM),
            pl.BlockSpec(memory_space=pltpu.VMEM),
        ),
        scratch_shapes=[
            pltpu.SemaphoreType.DMA,
            pltpu.SemaphoreType.DMA,
            pltpu.SemaphoreType.DMA,
            pltpu.SemaphoreType.DMA,
        ],
        compiler_params=pltpu.CompilerParams(
            collective_id=0, vmem_limit_bytes=VMEM_LIMIT
        ),
    )(x_bf16, assign2d)


def _moe_body(xg_ref, w1_ref, w2_ref, og_ref, acc_ref):
    fb = pl.program_id(1)
    n_fb = pl.num_programs(1)
    xg = xg_ref[0]
    w1 = w1_ref[0].astype(jnp.bfloat16)
    h = jnp.maximum(
        jnp.dot(xg, w1, preferred_element_type=jnp.float32), 0.0
    ).astype(jnp.bfloat16)
    w2 = w2_ref[0].astype(jnp.bfloat16)
    p = jnp.dot(h, w2, preferred_element_type=jnp.float32)

    @pl.when(fb == 0)
    def _():
        acc_ref[...] = p

    @pl.when(fb > 0)
    def _():
        acc_ref[...] += p

    @pl.when(fb == n_fb - 1)
    def _():
        og_ref[0] = acc_ref[...].astype(jnp.bfloat16)


def _moe(xg, w1, w2):
    n_fb = F // F_TILE
    return pl.pallas_call(
        _moe_body,
        grid=(E_LOCAL, n_fb),
        out_shape=jax.ShapeDtypeStruct((E_LOCAL, C, D), jnp.bfloat16),
        in_specs=[
            pl.BlockSpec((1, C, D), lambda e, fb: (e, 0, 0)),
            pl.BlockSpec((1, D, F_TILE), lambda e, fb: (e, 0, fb)),
            pl.BlockSpec((1, F_TILE, D), lambda e, fb: (e, fb, 0)),
        ],
        out_specs=pl.BlockSpec((1, C, D), lambda e, fb: (e, 0, 0)),
        scratch_shapes=[pltpu.VMEM((C, D), jnp.float32)],
        compiler_params=pltpu.CompilerParams(vmem_limit_bytes=VMEM_LIMIT),
    )(xg, w1, w2)


def _combine_body(in_ref, out_ref, s_send, s_recv):
    my_y, nbr = _neighbor()

    barrier = pltpu.get_barrier_semaphore()
    pl.semaphore_signal(barrier, inc=1, device_id=nbr,
                        device_id_type=pl.DeviceIdType.MESH)
    pl.semaphore_wait(barrier, 1)

    other0 = (1 - my_y) * T
    r = pltpu.make_async_remote_copy(
        src_ref=in_ref.at[pl.ds(other0, T)],
        dst_ref=out_ref,
        send_sem=s_send, recv_sem=s_recv,
        device_id=nbr, device_id_type=pl.DeviceIdType.MESH,
    )
    r.start()
    r.wait()

    mine0 = my_y * T
    out_ref[...] = out_ref[...] + in_ref[pl.ds(mine0, T), :]


def _combine(out_all):
    return pl.pallas_call(
        _combine_body,
        out_shape=jax.ShapeDtypeStruct((T, D), jnp.bfloat16),
        in_specs=[pl.BlockSpec(memory_space=pltpu.VMEM)],
        out_specs=pl.BlockSpec(memory_space=pltpu.VMEM),
        scratch_shapes=[
            pltpu.SemaphoreType.DMA,
            pltpu.SemaphoreType.DMA,
        ],
        compiler_params=pltpu.CompilerParams(
            collective_id=1, vmem_limit_bytes=VMEM_LIMIT
        ),
    )(out_all)


def kernel(x, assign, W1, W2):
    x_all, a2d = _exchange(x.astype(jnp.bfloat16), assign.reshape(A_ROWS, 128))
    assign_all = a2d.reshape(N_TOK)

    my_y = lax.axis_index("y")
    sort_idx = jnp.argsort(assign_all)
    sorted_a = assign_all[sort_idx]
    starts = jnp.searchsorted(sorted_a, jnp.arange(9, dtype=assign_all.dtype))
    e_ids = my_y * E_LOCAL + jnp.arange(E_LOCAL)
    start_e = starts[e_ids]
    end_e = starts[e_ids + 1]
    pos = start_e[:, None] + jnp.arange(C)[None, :]
    mask = pos < end_e[:, None]
    idx = sort_idx[jnp.minimum(pos, N_TOK - 1)]
    xg = x_all[idx]

    og = _moe(xg, W1, W2)

    flat_idx = jnp.where(mask, idx, N_TOK).reshape(-1)
    out_all = (
        jnp.zeros((N_TOK, D), jnp.bfloat16)
        .at[flat_idx]
        .set(og.reshape(-1, D), mode="drop")
    )

    out = _combine(out_all)
    return out.astype(jnp.float32)
